# baseline (speedup 1.0000x reference)
import functools

import jax
import jax.numpy as jnp
from jax import lax
from jax.experimental import pallas as pl
from jax.experimental.pallas import tpu as pltpu
from jax.experimental.pallas import tpu_sc as plsc

S = 16384
HL = 1024
HR = 256
W = HL + 2 * HR

NC = 2
NS = 16
NW = NC * NS
BW = S // NW
R = 64
NCH = BW // R


@functools.partial(
    pl.kernel,
    mesh=plsc.VectorSubcoreMesh(core_axis_name="c", subcore_axis_name="s"),
    out_type=jax.ShapeDtypeStruct((S, W), jnp.float32),
    scratch_types=[
        pltpu.VMEM((NCH, R), jnp.int32),
        pltpu.VMEM((NCH, R), jnp.int32),
        pltpu.VMEM((NCH, R), jnp.int32),
        pltpu.VMEM((R, HL), jnp.float32),
        pltpu.VMEM((R, HR), jnp.float32),
        pltpu.VMEM((R, HR), jnp.float32),
        pltpu.SemaphoreType.DMA,
    ],
)
def _sc_embed(lab_i_hbm, row_i_hbm, col_i_hbm, lab_w_hbm, row_w_hbm,
              col_w_hbm, out_hbm, lab_i, row_i, col_i, lab_v, row_v,
              col_v, sem):
    wid = lax.axis_index("s") * NC + lax.axis_index("c")
    pltpu.sync_copy(lab_i_hbm.at[wid], lab_i)
    pltpu.sync_copy(row_i_hbm.at[wid], row_i)
    pltpu.sync_copy(col_i_hbm.at[wid], col_i)

    def body(j, carry):
        base = wid * BW + j * R
        c1 = pltpu.async_copy(lab_w_hbm.at[lab_i.at[j]], lab_v, sem)
        c2 = pltpu.async_copy(row_w_hbm.at[row_i.at[j]], row_v, sem)
        c3 = pltpu.async_copy(col_w_hbm.at[col_i.at[j]], col_v, sem)
        c1.wait()
        c2.wait()
        c3.wait()
        pltpu.sync_copy(lab_v, out_hbm.at[pl.ds(base, R), pl.ds(0, HL)])
        pltpu.sync_copy(row_v, out_hbm.at[pl.ds(base, R), pl.ds(HL, HR)])
        pltpu.sync_copy(col_v, out_hbm.at[pl.ds(base, R), pl.ds(HL + HR, HR)])
        return carry

    lax.fori_loop(0, NCH, body, 0)


def kernel(label, label_logits, row_id, column_id, epoch, label_emb_w,
           row_emb_w, col_emb_w):
    del label_logits, epoch
    lab_i = label.astype(jnp.int32).reshape(NW, NCH, R)
    row_i = row_id.astype(jnp.int32).reshape(NW, NCH, R)
    col_i = column_id.astype(jnp.int32).reshape(NW, NCH, R)
    return _sc_embed(lab_i, row_i, col_i, label_emb_w, row_emb_w, col_emb_w)


# speedup vs baseline: 1.1990x; 1.1990x over previous
import functools

import jax
import jax.numpy as jnp
from jax import lax
from jax.experimental import pallas as pl
from jax.experimental.pallas import tpu as pltpu
from jax.experimental.pallas import tpu_sc as plsc

S = 16384
HL = 1024
HR = 256
W = HL + 2 * HR

NC = 2
NS = 16
NW = NC * NS
BW = S // NW
R = 32
NCH = BW // R


@functools.partial(
    pl.kernel,
    mesh=plsc.VectorSubcoreMesh(core_axis_name="c", subcore_axis_name="s"),
    out_type=jax.ShapeDtypeStruct((S, W), jnp.float32),
    scratch_types=[
        pltpu.VMEM((NCH, R), jnp.int32),
        pltpu.VMEM((NCH, R), jnp.int32),
        pltpu.VMEM((NCH, R), jnp.int32),
        pltpu.VMEM((R, W), jnp.float32),
        pltpu.VMEM((R, W), jnp.float32),
        pltpu.SemaphoreType.DMA,
        pltpu.SemaphoreType.DMA,
        pltpu.SemaphoreType.DMA,
        pltpu.SemaphoreType.DMA,
    ],
)
def _sc_embed(lab_i_hbm, row_i_hbm, col_i_hbm, lab_w_hbm, row_w_hbm,
              col_w_hbm, out_hbm, lab_i, row_i, col_i, buf0, buf1,
              gsem0, gsem1, ssem0, ssem1):
    wid = lax.axis_index("s") * NC + lax.axis_index("c")
    pltpu.sync_copy(lab_i_hbm.at[wid], lab_i)
    pltpu.sync_copy(row_i_hbm.at[wid], row_i)
    pltpu.sync_copy(col_i_hbm.at[wid], col_i)

    def gather(c, buf, sem):
        c1 = pltpu.async_copy(
            lab_w_hbm.at[lab_i.at[c]], buf.at[:, pl.ds(0, HL)], sem)
        c2 = pltpu.async_copy(
            row_w_hbm.at[row_i.at[c]], buf.at[:, pl.ds(HL, HR)], sem)
        c3 = pltpu.async_copy(
            col_w_hbm.at[col_i.at[c]], buf.at[:, pl.ds(HL + HR, HR)], sem)
        return c1, c2, c3

    def wait3(copies):
        for c in copies:
            c.wait()

    g0 = gather(0, buf0, gsem0)
    g1 = gather(1, buf1, gsem1)
    del g0, g1

    def body(g, carry):
        c0 = 2 * g
        c1 = c0 + 1
        d0 = pltpu.make_async_copy(
            lab_w_hbm.at[lab_i.at[c0]], buf0.at[:, pl.ds(0, HL)], gsem0)
        d0.wait()
        pltpu.make_async_copy(
            row_w_hbm.at[row_i.at[c0]], buf0.at[:, pl.ds(HL, HR)],
            gsem0).wait()
        pltpu.make_async_copy(
            col_w_hbm.at[col_i.at[c0]], buf0.at[:, pl.ds(HL + HR, HR)],
            gsem0).wait()
        s0 = pltpu.async_copy(
            buf0, out_hbm.at[pl.ds(wid * BW + c0 * R, R)], ssem0)

        pltpu.make_async_copy(
            lab_w_hbm.at[lab_i.at[c1]], buf1.at[:, pl.ds(0, HL)],
            gsem1).wait()
        pltpu.make_async_copy(
            row_w_hbm.at[row_i.at[c1]], buf1.at[:, pl.ds(HL, HR)],
            gsem1).wait()
        pltpu.make_async_copy(
            col_w_hbm.at[col_i.at[c1]], buf1.at[:, pl.ds(HL + HR, HR)],
            gsem1).wait()
        s1 = pltpu.async_copy(
            buf1, out_hbm.at[pl.ds(wid * BW + c1 * R, R)], ssem1)

        s0.wait()

        @pl.when(c0 + 2 < NCH)
        def _():
            gather(c0 + 2, buf0, gsem0)

        s1.wait()

        @pl.when(c1 + 2 < NCH)
        def _():
            gather(c1 + 2, buf1, gsem1)

        return carry

    lax.fori_loop(0, NCH // 2, body, 0)


def kernel(label, label_logits, row_id, column_id, epoch, label_emb_w,
           row_emb_w, col_emb_w):
    del label_logits, epoch
    lab_i = label.astype(jnp.int32).reshape(NW, NCH, R)
    row_i = row_id.astype(jnp.int32).reshape(NW, NCH, R)
    col_i = column_id.astype(jnp.int32).reshape(NW, NCH, R)
    return _sc_embed(lab_i, row_i, col_i, label_emb_w, row_emb_w, col_emb_w)
